# Initial kernel scaffold; baseline (speedup 1.0000x reference)
import jax
import jax.numpy as jnp
from jax.experimental import pallas as pl


def kernel(xyz, points, W1, b1, g1, be1, W2, b2, g2, be2, W3, b3, g_out, be_out):
    raise NotImplementedError("write your pallas kernel here")


# speedup vs baseline: 8.1607x; 8.1607x over previous
import functools

import jax
import jax.numpy as jnp
from jax import lax
from jax.experimental import pallas as pl
from jax.experimental.pallas import tpu as pltpu
from jax.experimental.pallas import tpu_sc as plsc

_EPS = 1e-5
_K = 16
_NPAD = 5120
_MB = 200
_MN = 1000
_HI = jax.lax.Precision.HIGHEST


def _knn_body(n, xt_ref, q_ref, idx_ref):
    b = pl.program_id(0)
    xt = xt_ref[0]
    q = q_ref[0]
    sx = jnp.sum(xt * xt, axis=0, keepdims=True)
    sq = jnp.sum(q * q, axis=1, keepdims=True)
    d0 = lax.dot_general(q, xt, (((1,), (0,)), ((), ())),
                         preferred_element_type=jnp.float32)
    d = -2.0 * d0 + sq + sx
    iota = lax.broadcasted_iota(jnp.int32, (_MB, _NPAD), 1)
    cols = []
    for _ in range(_K):
        m = jnp.min(d, axis=1, keepdims=True)
        cand = jnp.where(d == m, iota, _NPAD)
        amin = jnp.min(cand, axis=1, keepdims=True)
        cols.append(amin + b * n)
        d = jnp.where(iota == amin, jnp.float32(jnp.inf), d)
    idx_ref[0] = jnp.concatenate(cols, axis=1)


def _knn(xyz_t_pad, xyz):
    b_sz, n, _ = xyz.shape
    return pl.pallas_call(
        functools.partial(_knn_body, n),
        grid=(b_sz, n // _MB),
        in_specs=[
            pl.BlockSpec((1, 3, _NPAD), lambda b, i: (b, 0, 0)),
            pl.BlockSpec((1, _MB, 3), lambda b, i: (b, i, 0)),
        ],
        out_specs=pl.BlockSpec((1, _MB, _K), lambda b, i: (b, i, 0)),
        out_shape=jax.ShapeDtypeStruct((b_sz, n, _K), jnp.int32),
    )(xyz_t_pad, xyz)


def _sc_gather(table, idx2d):
    info = plsc.get_sparse_core_info()
    nw = info.num_cores * info.num_subcores
    nrow = idx2d.shape[0]
    cpw = nrow // nw
    d = table.shape[1]
    mesh = plsc.VectorSubcoreMesh(core_axis_name="c", subcore_axis_name="s")

    @functools.partial(
        pl.kernel, mesh=mesh,
        compiler_params=pltpu.CompilerParams(use_tc_tiling_on_sc=False),
        out_type=jax.ShapeDtypeStruct((nrow * 128, d), jnp.float32),
        scratch_types=[
            pltpu.VMEM((cpw, 128), jnp.int32),
            pltpu.VMEM((128, d), jnp.float32),
            pltpu.SemaphoreType.DMA,
        ],
    )
    def k(table_hbm, idx_hbm, out_hbm, idx_v, rows_v, sem):
        wid = lax.axis_index("s") * info.num_cores + lax.axis_index("c")
        pltpu.sync_copy(idx_hbm.at[pl.ds(wid * cpw, cpw)], idx_v)

        def body(i, carry):
            pltpu.async_copy(table_hbm.at[idx_v.at[i]], rows_v, sem).wait()
            pltpu.sync_copy(rows_v, out_hbm.at[pl.ds((wid * cpw + i) * 128, 128)])
            return carry

        lax.fori_loop(0, cpw, body, 0)

    return k(table, idx2d)


_DN = (((1,), (0,)), ((), ()))


def _dot(x, w):
    return lax.dot_general(x, w, _DN, preferred_element_type=jnp.float32,
                           precision=_HI)


def _rel(gk, xq, sel):
    return gk * sel - xq


def _sel32():
    l = lax.broadcasted_iota(jnp.int32, (1, 32), 1)
    return jnp.where((l >= 16) & (l < 19), 1.0, 0.0)


def _acc_stats(ref, s, q):
    st = jnp.concatenate([s, q], axis=0)

    @pl.when(pl.program_id(0) == 0)
    def _():
        ref[...] = jnp.zeros_like(ref)

    ref[...] += st


def _row_sums(xs):
    total = xs[0]
    sq = xs[0] * xs[0]
    for x in xs[1:]:
        total = total + x
        sq = sq + x * x
    ones = jnp.ones((1, total.shape[0]), jnp.float32)
    return _dot(ones, total), _dot(ones, sq)


def _pass_a_body(g_ref, xq_ref, w1_ref, b1_ref, st_ref):
    xq = xq_ref[...]
    sel = _sel32()
    h1s = [jnp.maximum(_dot(_rel(g_ref[k], xq, sel), w1_ref[...])
                       + b1_ref[...], 0.0) for k in range(_K)]
    s, q = _row_sums(h1s)
    _acc_stats(st_ref, s, q)


def _pass_b_body(g_ref, xq_ref, w1_ref, b1_ref, w2_ref, b2_ref, st_ref):
    xq = xq_ref[...]
    sel = _sel32()
    h2s = []
    for k in range(_K):
        h1 = jnp.maximum(_dot(_rel(g_ref[k], xq, sel), w1_ref[...])
                         + b1_ref[...], 0.0)
        h2s.append(jnp.maximum(_dot(h1, w2_ref[...]) + b2_ref[...], 0.0))
    s, q = _row_sums(h2s)
    _acc_stats(st_ref, s, q)


def _pass_c_body(g_ref, xq_ref, w1_ref, b1_ref, w2_ref, b2_ref,
                 w3_ref, b3_ref, e_ref, f_ref, out_ref, st_ref):
    xq = xq_ref[...]
    sel = _sel32()
    o = None
    for k in range(_K):
        gk = g_ref[k]
        h1 = jnp.maximum(_dot(_rel(gk, xq, sel), w1_ref[...])
                         + b1_ref[...], 0.0)
        h2 = jnp.maximum(_dot(h1, w2_ref[...]) + b2_ref[...], 0.0)
        w = _dot(h2, w3_ref[...]) + b3_ref[...]
        p_rep = _dot(gk[:, 0:16], e_ref[...])
        acc = _dot(w * p_rep, f_ref[...])
        o = acc if o is None else jnp.maximum(o, acc)
    out_ref[...] = o
    ones = jnp.ones((1, o.shape[0]), jnp.float32)
    _acc_stats(st_ref, _dot(ones, o), _dot(ones, o * o))


def _stats_specs(c):
    return pl.BlockSpec((2, c), lambda i: (0, 0)), jax.ShapeDtypeStruct((2, c), jnp.float32)


def _g_spec():
    return pl.BlockSpec((_K, _MN, 32), lambda i: (0, i, 0))


def _row_spec(rows, cols):
    return pl.BlockSpec((rows, cols), lambda i: (i, 0))


def _full_spec(shape):
    return pl.BlockSpec(shape, lambda i: tuple(0 for _ in shape))


def _pass_a(gat3, xq32, w1, b1):
    sspec, sshape = _stats_specs(32)
    return pl.pallas_call(
        _pass_a_body, grid=(gat3.shape[1] // _MN,),
        in_specs=[_g_spec(), _row_spec(_MN, 32),
                  _full_spec((32, 32)), _full_spec((1, 32))],
        out_specs=sspec, out_shape=sshape,
    )(gat3, xq32, w1, b1)


def _pass_b(gat3, xq32, w1, b1, w2, b2):
    sspec, sshape = _stats_specs(32)
    return pl.pallas_call(
        _pass_b_body, grid=(gat3.shape[1] // _MN,),
        in_specs=[_g_spec(), _row_spec(_MN, 32),
                  _full_spec((32, 32)), _full_spec((1, 32)),
                  _full_spec((32, 32)), _full_spec((1, 32))],
        out_specs=sspec, out_shape=sshape,
    )(gat3, xq32, w1, b1, w2, b2)


def _pass_c(gat3, xq32, w1, b1, w2, b2, w3, b3):
    nq = gat3.shape[1]
    sspec, sshape = _stats_specs(8)
    l128 = jnp.arange(128, dtype=jnp.int32)
    e = (l128[None, :] // 8 == jnp.arange(16, dtype=jnp.int32)[:, None]
         ).astype(jnp.float32)
    f = (l128[:, None] % 8 == jnp.arange(8, dtype=jnp.int32)[None, :]
         ).astype(jnp.float32)
    return pl.pallas_call(
        _pass_c_body, grid=(nq // _MN,),
        in_specs=[_g_spec(), _row_spec(_MN, 32),
                  _full_spec((32, 32)), _full_spec((1, 32)),
                  _full_spec((32, 32)), _full_spec((1, 32)),
                  _full_spec((32, 128)), _full_spec((1, 128)),
                  _full_spec((16, 128)), _full_spec((128, 8))],
        out_specs=[_row_spec(_MN, 8), sspec],
        out_shape=[jax.ShapeDtypeStruct((nq, 8), jnp.float32), sshape],
    )(gat3, xq32, w1, b1, w2, b2, w3, b3, e, f)


def _final_body(n, x_ref, st_ref, g_ref, be_ref, out_ref):
    x = x_ref[...]
    m = st_ref[0:1, :] / n
    v = st_ref[1:2, :] / n - m * m
    out_ref[...] = g_ref[...] * (x - m) / jnp.sqrt(v + _EPS) + be_ref[...]


def _final(out_pre, st, g, be):
    bn = out_pre.shape[0]
    return pl.pallas_call(
        functools.partial(_final_body, float(bn)),
        grid=(1,),
        in_specs=[_row_spec(bn, 8), _full_spec((2, 8)),
                  _full_spec((1, 8)), _full_spec((1, 8))],
        out_specs=_row_spec(bn, 8),
        out_shape=jax.ShapeDtypeStruct((bn, 8), jnp.float32),
    )(out_pre, st, g, be)


def _fold(st, n, g, be, w_next, b_next):
    m = st[0] / n
    v = st[1] / n - m * m
    a = g / jnp.sqrt(v + _EPS)
    c = be - m * a
    return a[:, None] * w_next, c @ w_next + b_next


def kernel(xyz, points, W1, b1, g1, be1, W2, b2, g2, be2, W3, b3, g_out, be_out):
    b_sz, n, _ = xyz.shape
    bn = b_sz * n
    ne = bn * _K

    pad = jnp.full((b_sz, _NPAD - n, 3), 1e6, jnp.float32)
    xyz_t_pad = jnp.concatenate([xyz, pad], axis=1).transpose(0, 2, 1)
    idx = _knn(xyz_t_pad, xyz)

    table = jnp.concatenate(
        [points, xyz, jnp.zeros((b_sz, n, 13), jnp.float32)], axis=-1
    ).reshape(bn, 32)
    npad_idx = ((ne + 4095) // 4096) * 4096
    idx_flat = jnp.concatenate(
        [jnp.transpose(idx, (2, 0, 1)).reshape(-1),
         jnp.zeros((npad_idx - ne,), jnp.int32)])
    gat = _sc_gather(table, idx_flat.reshape(npad_idx // 128, 128))[:ne]
    gat3 = gat.reshape(_K, bn, 32)

    xq32 = jnp.concatenate(
        [jnp.zeros((bn, 16), jnp.float32), xyz.reshape(bn, 3),
         jnp.zeros((bn, 13), jnp.float32)], axis=1)
    w1g = jnp.zeros((32, 32), jnp.float32).at[16:19, :].set(W1)
    b1r, b3r = b1[None, :], b3[None, :]

    st1 = _pass_a(gat3, xq32, w1g, b1r)
    w2f, b2f = _fold(st1, float(ne), g1, be1, W2, b2)
    st2 = _pass_b(gat3, xq32, w1g, b1r, w2f, b2f[None, :])
    w3f, b3f = _fold(st2, float(ne), g2, be2, W3, b3r[0])
    out_pre, st3 = _pass_c(gat3, xq32, w1g, b1r, w2f, b2f[None, :],
                           w3f, b3f[None, :])
    out = _final(out_pre, st3, g_out[None, :], be_out[None, :])
    return out.reshape(b_sz, n, 8)


# speedup vs baseline: 13.4850x; 1.6524x over previous
import functools

import jax
import jax.numpy as jnp
from jax import lax
from jax.experimental import pallas as pl
from jax.experimental.pallas import tpu as pltpu
from jax.experimental.pallas import tpu_sc as plsc

_EPS = 1e-5
_K = 16
_NPAD = 5120
_MB = 200
_MN = 1000
_HI = jax.lax.Precision.HIGHEST


def _knn_body(n, xt_ref, q_ref, idx_ref):
    b = pl.program_id(0)
    xt = xt_ref[0]
    q = q_ref[0]
    sx = jnp.sum(xt * xt, axis=0, keepdims=True)
    sq = jnp.sum(q * q, axis=1, keepdims=True)
    d0 = lax.dot_general(q, xt, (((1,), (0,)), ((), ())),
                         preferred_element_type=jnp.float32)
    d = -2.0 * d0 + sq + sx
    iota = lax.broadcasted_iota(jnp.int32, (_MB, _NPAD), 1)
    cols = []
    for _ in range(_K):
        m = jnp.min(d, axis=1, keepdims=True)
        cand = jnp.where(d == m, iota, _NPAD)
        amin = jnp.min(cand, axis=1, keepdims=True)
        cols.append(amin + b * n)
        d = jnp.where(iota == amin, jnp.float32(jnp.inf), d)
    idx_ref[0] = jnp.concatenate(cols, axis=1)


def _knn(xyz_t_pad, xyz):
    b_sz, n, _ = xyz.shape
    return pl.pallas_call(
        functools.partial(_knn_body, n),
        grid=(b_sz, n // _MB),
        in_specs=[
            pl.BlockSpec((1, 3, _NPAD), lambda b, i: (b, 0, 0)),
            pl.BlockSpec((1, _MB, 3), lambda b, i: (b, i, 0)),
        ],
        out_specs=pl.BlockSpec((1, _MB, _K), lambda b, i: (b, i, 0)),
        out_shape=jax.ShapeDtypeStruct((b_sz, n, _K), jnp.int32),
    )(xyz_t_pad, xyz)


def _sc_gather(table, idx2d):
    info = plsc.get_sparse_core_info()
    nw = info.num_cores * info.num_subcores
    nrow = idx2d.shape[0]
    cpw = nrow // nw
    d = table.shape[1]
    mesh = plsc.VectorSubcoreMesh(core_axis_name="c", subcore_axis_name="s")

    @functools.partial(
        pl.kernel, mesh=mesh,
        compiler_params=pltpu.CompilerParams(use_tc_tiling_on_sc=False),
        out_type=jax.ShapeDtypeStruct((nrow * 128, d), jnp.float32),
        scratch_types=[
            pltpu.VMEM((cpw, 128), jnp.int32),
            pltpu.VMEM((128, d), jnp.float32),
            pltpu.SemaphoreType.DMA,
        ],
    )
    def k(table_hbm, idx_hbm, out_hbm, idx_v, rows_v, sem):
        wid = lax.axis_index("s") * info.num_cores + lax.axis_index("c")
        pltpu.sync_copy(idx_hbm.at[pl.ds(wid * cpw, cpw)], idx_v)

        def body(i, carry):
            pltpu.async_copy(table_hbm.at[idx_v.at[i]], rows_v, sem).wait()
            pltpu.sync_copy(rows_v, out_hbm.at[pl.ds((wid * cpw + i) * 128, 128)])
            return carry

        lax.fori_loop(0, cpw, body, 0)

    return k(table, idx2d)


_DN = (((1,), (0,)), ((), ()))


def _dot(x, w):
    return lax.dot_general(x, w, _DN, preferred_element_type=jnp.float32)


def _prep(g_ref, xq_ref):
    g = g_ref[...].reshape(_K * _MN, 32)
    xq = jnp.broadcast_to(xq_ref[...][None], (_K, _MN, 32)).reshape(_K * _MN, 32)
    l = lax.broadcasted_iota(jnp.int32, (1, 32), 1)
    sel = jnp.where((l >= 16) & (l < 19), 1.0, 0.0)
    return g, g * sel - xq


def _acc_stats(ref, x):
    ones = jnp.ones((1, x.shape[0]), jnp.float32)
    st = jnp.concatenate([_dot(ones, x), _dot(ones, x * x)], axis=0)

    @pl.when(pl.program_id(0) == 0)
    def _():
        ref[...] = jnp.zeros_like(ref)

    ref[...] += st


def _pass_a_body(g_ref, xq_ref, w1_ref, b1_ref, st_ref):
    _, rel = _prep(g_ref, xq_ref)
    h1 = jnp.maximum(_dot(rel, w1_ref[...]) + b1_ref[...], 0.0)
    _acc_stats(st_ref, h1)


def _pass_b_body(g_ref, xq_ref, w1_ref, b1_ref, w2_ref, b2_ref, st_ref):
    _, rel = _prep(g_ref, xq_ref)
    h1 = jnp.maximum(_dot(rel, w1_ref[...]) + b1_ref[...], 0.0)
    h2 = jnp.maximum(_dot(h1, w2_ref[...]) + b2_ref[...], 0.0)
    _acc_stats(st_ref, h2)


def _pass_c_body(g_ref, xq_ref, w1_ref, b1_ref, w2_ref, b2_ref,
                 w3_ref, b3_ref, e_ref, f_ref, out_ref, st_ref):
    g, rel = _prep(g_ref, xq_ref)
    h1 = jnp.maximum(_dot(rel, w1_ref[...]) + b1_ref[...], 0.0)
    h2 = jnp.maximum(_dot(h1, w2_ref[...]) + b2_ref[...], 0.0)
    w = _dot(h2, w3_ref[...]) + b3_ref[...]
    p_rep = lax.dot_general(g[:, 0:16], e_ref[...], _DN,
                            preferred_element_type=jnp.float32, precision=_HI)
    acc = lax.dot_general(w * p_rep, f_ref[...], _DN,
                          preferred_element_type=jnp.float32, precision=_HI)
    acc3 = acc.reshape(_K, _MN, 8)
    o = acc3[0]
    for k in range(1, _K):
        o = jnp.maximum(o, acc3[k])
    out_ref[...] = o
    _acc_stats(st_ref, o)


def _stats_specs(c):
    return pl.BlockSpec((2, c), lambda i: (0, 0)), jax.ShapeDtypeStruct((2, c), jnp.float32)


def _g_spec():
    return pl.BlockSpec((_K, _MN, 32), lambda i: (0, i, 0))


def _row_spec(rows, cols):
    return pl.BlockSpec((rows, cols), lambda i: (i, 0))


def _full_spec(shape):
    return pl.BlockSpec(shape, lambda i: tuple(0 for _ in shape))


def _pass_a(gat3, xq32, w1, b1):
    sspec, sshape = _stats_specs(32)
    return pl.pallas_call(
        _pass_a_body, grid=(gat3.shape[1] // _MN,),
        in_specs=[_g_spec(), _row_spec(_MN, 32),
                  _full_spec((32, 32)), _full_spec((1, 32))],
        out_specs=sspec, out_shape=sshape,
    )(gat3, xq32, w1, b1)


def _pass_b(gat3, xq32, w1, b1, w2, b2):
    sspec, sshape = _stats_specs(32)
    return pl.pallas_call(
        _pass_b_body, grid=(gat3.shape[1] // _MN,),
        in_specs=[_g_spec(), _row_spec(_MN, 32),
                  _full_spec((32, 32)), _full_spec((1, 32)),
                  _full_spec((32, 32)), _full_spec((1, 32))],
        out_specs=sspec, out_shape=sshape,
    )(gat3, xq32, w1, b1, w2, b2)


def _pass_c(gat3, xq32, w1, b1, w2, b2, w3, b3):
    nq = gat3.shape[1]
    sspec, sshape = _stats_specs(8)
    l128 = jnp.arange(128, dtype=jnp.int32)
    e = (l128[None, :] // 8 == jnp.arange(16, dtype=jnp.int32)[:, None]
         ).astype(jnp.float32)
    f = (l128[:, None] % 8 == jnp.arange(8, dtype=jnp.int32)[None, :]
         ).astype(jnp.float32)
    return pl.pallas_call(
        _pass_c_body, grid=(nq // _MN,),
        in_specs=[_g_spec(), _row_spec(_MN, 32),
                  _full_spec((32, 32)), _full_spec((1, 32)),
                  _full_spec((32, 32)), _full_spec((1, 32)),
                  _full_spec((32, 128)), _full_spec((1, 128)),
                  _full_spec((16, 128)), _full_spec((128, 8))],
        out_specs=[_row_spec(_MN, 8), sspec],
        out_shape=[jax.ShapeDtypeStruct((nq, 8), jnp.float32), sshape],
    )(gat3, xq32, w1, b1, w2, b2, w3, b3, e, f)


def _final_body(n, x_ref, st_ref, g_ref, be_ref, out_ref):
    x = x_ref[...]
    m = st_ref[0:1, :] / n
    v = st_ref[1:2, :] / n - m * m
    out_ref[...] = g_ref[...] * (x - m) / jnp.sqrt(v + _EPS) + be_ref[...]


def _final(out_pre, st, g, be):
    bn = out_pre.shape[0]
    return pl.pallas_call(
        functools.partial(_final_body, float(bn)),
        grid=(1,),
        in_specs=[_row_spec(bn, 8), _full_spec((2, 8)),
                  _full_spec((1, 8)), _full_spec((1, 8))],
        out_specs=_row_spec(bn, 8),
        out_shape=jax.ShapeDtypeStruct((bn, 8), jnp.float32),
    )(out_pre, st, g, be)


def _fold(st, n, g, be, w_next, b_next):
    m = st[0] / n
    v = st[1] / n - m * m
    a = g / jnp.sqrt(v + _EPS)
    c = be - m * a
    return a[:, None] * w_next, c @ w_next + b_next


def kernel(xyz, points, W1, b1, g1, be1, W2, b2, g2, be2, W3, b3, g_out, be_out):
    b_sz, n, _ = xyz.shape
    bn = b_sz * n
    ne = bn * _K

    pad = jnp.full((b_sz, _NPAD - n, 3), 1e6, jnp.float32)
    xyz_t_pad = jnp.concatenate([xyz, pad], axis=1).transpose(0, 2, 1)
    idx = _knn(xyz_t_pad, xyz)

    table = jnp.concatenate(
        [points, xyz, jnp.zeros((b_sz, n, 13), jnp.float32)], axis=-1
    ).reshape(bn, 32)
    npad_idx = ((ne + 4095) // 4096) * 4096
    idx_flat = jnp.concatenate(
        [jnp.transpose(idx, (2, 0, 1)).reshape(-1),
         jnp.zeros((npad_idx - ne,), jnp.int32)])
    gat = _sc_gather(table, idx_flat.reshape(npad_idx // 128, 128))[:ne]
    gat3 = gat.reshape(_K, bn, 32)

    xq32 = jnp.concatenate(
        [jnp.zeros((bn, 16), jnp.float32), xyz.reshape(bn, 3),
         jnp.zeros((bn, 13), jnp.float32)], axis=1)
    w1g = jnp.zeros((32, 32), jnp.float32).at[16:19, :].set(W1)
    b1r, b3r = b1[None, :], b3[None, :]

    st1 = _pass_a(gat3, xq32, w1g, b1r)
    w2f, b2f = _fold(st1, float(ne), g1, be1, W2, b2)
    st2 = _pass_b(gat3, xq32, w1g, b1r, w2f, b2f[None, :])
    w3f, b3f = _fold(st2, float(ne), g2, be2, W3, b3r[0])
    out_pre, st3 = _pass_c(gat3, xq32, w1g, b1r, w2f, b2f[None, :],
                           w3f, b3f[None, :])
    out = _final(out_pre, st3, g_out[None, :], be_out[None, :])
    return out.reshape(b_sz, n, 8)
